# Initial kernel scaffold; baseline (speedup 1.0000x reference)
import jax
import jax.numpy as jnp
from jax.experimental import pallas as pl


def kernel(x, edge_index_r0, edge_index_r1, edge_index_r2, edge_index_r3, w, coeff_mat, W_self, bias):
    raise NotImplementedError("write your pallas kernel here")


# speedup vs baseline: 1.9354x; 1.9354x over previous
import functools

import jax
import jax.numpy as jnp
from jax import lax
from jax.experimental import pallas as pl
from jax.experimental.pallas import tpu as pltpu
from jax.experimental.pallas import tpu_sc as plsc

N_NODES = 10000
N_REL = 4
D_IN = 256
D_OUT = 256
DH = 144
CH = 128
E = 40000
LANES = 16
NC = 2
NS = 16
CHUNK = 64
CHUNKS_PER_TILE = 40
E_TILE = CHUNKS_PER_TILE * CHUNK
E_PAD = NS * E_TILE
ACC_ROWS = 10016
OUT_UNIT = 40
N_UNITS = N_NODES // OUT_UNIT


def _sc_body(xs_hbm, src_hbm, dst_hbm, out_hbm,
             acc_sh, src_all, ldst_all, rows2, zrow_v, sems):
    i32 = jnp.int32
    cid = lax.axis_index("c").astype(i32)
    sid = lax.axis_index("s").astype(i32)

    @pl.loop(i32(0), i32(DH // LANES))
    def _zb(j):
        j = j.astype(jnp.int32)
        for i in range(8):
            zrow_v[i, pl.ds(j * LANES, LANES)] = jnp.zeros((LANES,), jnp.float32)

    def start_gather(j, b):
        pltpu.async_copy(
            xs_hbm.at[cid].at[src_all.at[pl.ds(j * i32(CHUNK), CHUNK)]],
            rows2.at[i32(b)], sems.at[i32(b)])

    def wait_gather(b):
        pltpu.make_async_copy(
            xs_hbm.at[i32(0), pl.ds(i32(0), CHUNK)],
            rows2.at[i32(b)], sems.at[i32(b)]).wait()

    for r in range(N_REL):
        r = i32(r)

        @pl.loop(i32(0), i32((ACC_ROWS // 8 + NS - 1) // NS))
        def _z(k):
            zu = sid + k.astype(jnp.int32) * i32(NS)

            @pl.when(zu < i32(ACC_ROWS // 8))
            def _():
                pltpu.sync_copy(zrow_v, acc_sh.at[pl.ds(zu * 8, 8)])

        ebase = sid * i32(E_TILE)
        pltpu.sync_copy(src_hbm.at[r, pl.ds(ebase, E_TILE)], src_all)

        @pl.loop(i32(0), i32(CHUNKS_PER_TILE))
        def _ld(j):
            j = j.astype(jnp.int32)
            pltpu.sync_copy(dst_hbm.at[r, pl.ds(ebase + j * CHUNK, CHUNK)],
                            ldst_all.at[j])
            for i in range(CHUNK // LANES):
                d = ldst_all[j, pl.ds(i * LANES, LANES)]
                ok = d < i32(N_NODES)
                ldst_all[j, pl.ds(i * LANES, LANES)] = jnp.where(
                    ok, d, i32(N_NODES) + sid)

        plsc.subcore_barrier()

        start_gather(i32(0), 0)
        start_gather(i32(1), 1)

        @pl.loop(i32(0), i32(CHUNKS_PER_TILE), step=i32(2))
        def _e(j):
            j = j.astype(jnp.int32)
            for b in range(2):
                jj = j + i32(b)
                wait_gather(b)
                pltpu.sync_copy(rows2.at[i32(b)], acc_sh.at[ldst_all.at[jj]],
                                add=True)

                @pl.when(jj + i32(2) < i32(CHUNKS_PER_TILE))
                def _():
                    start_gather(jj + i32(2), b)

        plsc.subcore_barrier()

        @pl.loop(i32(0), i32((N_UNITS + NS - 1) // NS))
        def _w(k):
            u = sid + k.astype(jnp.int32) * i32(NS)

            @pl.when(u < i32(N_UNITS))
            def _():
                pltpu.sync_copy(
                    acc_sh.at[pl.ds(u * OUT_UNIT, OUT_UNIT)],
                    out_hbm.at[cid, r, pl.ds(u * OUT_UNIT, OUT_UNIT)])

        plsc.subcore_barrier()


@functools.partial(jax.jit, static_argnames=())
def _sc_aggregate(xs, src, dst):
    mesh = plsc.VectorSubcoreMesh(core_axis_name="c", subcore_axis_name="s")
    f = pl.kernel(
        _sc_body,
        out_type=jax.ShapeDtypeStruct((NC, N_REL, N_NODES, DH), jnp.float32),
        mesh=mesh,
        scratch_types=[
            pltpu.VMEM_SHARED((ACC_ROWS, DH), jnp.float32),
            pltpu.VMEM((E_TILE,), jnp.int32),
            pltpu.VMEM((CHUNKS_PER_TILE, CHUNK), jnp.int32),
            pltpu.VMEM((2, CHUNK, DH), jnp.float32),
            pltpu.VMEM((8, DH), jnp.float32),
            pltpu.SemaphoreType.DMA((2,)),
        ],
        compiler_params=pltpu.CompilerParams(use_tc_tiling_on_sc=False),
    )
    return f(xs, src, dst)


def _wcat_body(coeff_ref, w2d_ref, wself_ref, o_ref):
    wall = jnp.dot(coeff_ref[...], w2d_ref[...],
                   preferred_element_type=jnp.float32,
                   precision=jax.lax.Precision.HIGHEST)
    o_ref[: N_REL * D_IN, :] = wall.reshape(N_REL * D_IN, D_OUT)
    o_ref[N_REL * D_IN:, :] = wself_ref[...]


def _combine_wcat(coeff_mat, w2d, W_self):
    return pl.pallas_call(
        _wcat_body,
        out_shape=jax.ShapeDtypeStruct(((N_REL + 1) * D_IN, D_OUT), jnp.float32),
    )(coeff_mat, w2d, W_self)


def _z0(i):
    return i * 0


BM = 400


def _tc_body(s_ref, x_ref, wcat_ref, bias_ref, o_ref):
    parts = []
    for r in range(N_REL):
        cnt = s_ref[0, r, :, CH:CH + 1]
        sm = jnp.concatenate(
            [s_ref[0, r, :, :CH], s_ref[1, r, :, :CH]], axis=1)
        parts.append(jnp.where(cnt > 0, sm / jnp.maximum(cnt, 1.0), 0.0))
    parts.append(x_ref[...])
    xin = jnp.concatenate(parts, axis=1)
    acc = jnp.dot(xin, wcat_ref[...], preferred_element_type=jnp.float32)
    o_ref[...] = jnp.maximum(acc + bias_ref[...], 0.0)


def _tc_combine(s, x, wcat, bias2d):
    grid = (N_NODES // BM,)
    return pl.pallas_call(
        _tc_body,
        grid=grid,
        in_specs=[
            pl.BlockSpec((NC, N_REL, BM, DH),
                         lambda i: (_z0(i), _z0(i), i, _z0(i))),
            pl.BlockSpec((BM, D_IN), lambda i: (i, _z0(i))),
            pl.BlockSpec(((N_REL + 1) * D_IN, D_OUT),
                         lambda i: (_z0(i), _z0(i))),
            pl.BlockSpec((1, D_OUT), lambda i: (_z0(i), _z0(i))),
        ],
        out_specs=pl.BlockSpec((BM, D_OUT), lambda i: (i, _z0(i))),
        out_shape=jax.ShapeDtypeStruct((N_NODES, D_OUT), jnp.float32),
    )(s, x, wcat, bias2d)


def kernel(x, edge_index_r0, edge_index_r1, edge_index_r2, edge_index_r3,
           w, coeff_mat, W_self, bias):
    x = x.astype(jnp.float32)
    ones = jnp.ones((N_NODES, 1), jnp.float32)
    zpad = jnp.zeros((N_NODES, DH - CH - 1), jnp.float32)
    xs = jnp.stack([
        jnp.concatenate([x[:, :CH], ones, zpad], axis=1),
        jnp.concatenate([x[:, CH:], ones, zpad], axis=1),
    ])

    srcs, dsts = [], []
    for e in (edge_index_r0, edge_index_r1, edge_index_r2, edge_index_r3):
        src = e[0].astype(jnp.int32)
        dst = e[1].astype(jnp.int32)
        srcs.append(jnp.concatenate(
            [src, jnp.zeros((E_PAD - E,), jnp.int32)]))
        dsts.append(jnp.concatenate(
            [dst, jnp.full((E_PAD - E,), N_NODES, jnp.int32)]))
    src = jnp.stack(srcs)
    dst = jnp.stack(dsts)

    s = _sc_aggregate(xs, src, dst)

    w2d = w.astype(jnp.float32).reshape(w.shape[0], D_IN * D_OUT)
    wcat = _combine_wcat(coeff_mat.astype(jnp.float32), w2d,
                         W_self.astype(jnp.float32))
    bias2d = bias.astype(jnp.float32).reshape(1, D_OUT)
    return _tc_combine(s, x, wcat, bias2d)


# speedup vs baseline: 1.9922x; 1.0294x over previous
import functools

import jax
import jax.numpy as jnp
from jax import lax
from jax.experimental import pallas as pl
from jax.experimental.pallas import tpu as pltpu
from jax.experimental.pallas import tpu_sc as plsc

N_NODES = 10000
N_REL = 4
D_IN = 256
D_OUT = 256
DH = 144
CH = 128
E = 40000
LANES = 16
NC = 2
NS = 16
CHUNK = 80
CHUNKS_PER_TILE = 32
E_TILE = CHUNKS_PER_TILE * CHUNK
E_PAD = NS * E_TILE
ACC_ROWS = 10016
OUT_UNIT = 40
N_UNITS = N_NODES // OUT_UNIT


def _sc_body(xs_hbm, src_hbm, dst_hbm, out_hbm,
             acc_sh, src_all, ldst_all, rows2, zrow_v, sems):
    i32 = jnp.int32
    cid = lax.axis_index("c").astype(i32)
    sid = lax.axis_index("s").astype(i32)

    @pl.loop(i32(0), i32(DH // LANES))
    def _zb(j):
        j = j.astype(jnp.int32)
        for i in range(8):
            zrow_v[i, pl.ds(j * LANES, LANES)] = jnp.zeros((LANES,), jnp.float32)

    def start_gather(j, b):
        pltpu.async_copy(
            xs_hbm.at[cid].at[src_all.at[pl.ds(j * i32(CHUNK), CHUNK)]],
            rows2.at[i32(b)], sems.at[i32(b)])

    def wait_gather(b):
        pltpu.make_async_copy(
            xs_hbm.at[i32(0), pl.ds(i32(0), CHUNK)],
            rows2.at[i32(b)], sems.at[i32(b)]).wait()

    for r in range(N_REL):
        r = i32(r)

        @pl.loop(i32(0), i32((ACC_ROWS // 8 + NS - 1) // NS))
        def _z(k):
            zu = sid + k.astype(jnp.int32) * i32(NS)

            @pl.when(zu < i32(ACC_ROWS // 8))
            def _():
                pltpu.sync_copy(zrow_v, acc_sh.at[pl.ds(zu * 8, 8)])

        ebase = sid * i32(E_TILE)
        pltpu.sync_copy(src_hbm.at[r, pl.ds(ebase, E_TILE)], src_all)

        @pl.loop(i32(0), i32(CHUNKS_PER_TILE))
        def _ld(j):
            j = j.astype(jnp.int32)
            pltpu.sync_copy(dst_hbm.at[r, pl.ds(ebase + j * CHUNK, CHUNK)],
                            ldst_all.at[j])
            for i in range(CHUNK // LANES):
                d = ldst_all[j, pl.ds(i * LANES, LANES)]
                ok = d < i32(N_NODES)
                ldst_all[j, pl.ds(i * LANES, LANES)] = jnp.where(
                    ok, d, i32(N_NODES) + sid)

        plsc.subcore_barrier()

        start_gather(i32(0), 0)
        start_gather(i32(1), 1)

        @pl.loop(i32(0), i32(CHUNKS_PER_TILE), step=i32(2))
        def _e(j):
            j = j.astype(jnp.int32)
            for b in range(2):
                jj = j + i32(b)
                wait_gather(b)
                pltpu.sync_copy(rows2.at[i32(b)], acc_sh.at[ldst_all.at[jj]],
                                add=True)

                @pl.when(jj + i32(2) < i32(CHUNKS_PER_TILE))
                def _():
                    start_gather(jj + i32(2), b)

        plsc.subcore_barrier()

        @pl.loop(i32(0), i32((N_UNITS + NS - 1) // NS))
        def _w(k):
            u = sid + k.astype(jnp.int32) * i32(NS)

            @pl.when(u < i32(N_UNITS))
            def _():
                pltpu.sync_copy(
                    acc_sh.at[pl.ds(u * OUT_UNIT, OUT_UNIT)],
                    out_hbm.at[cid, r, pl.ds(u * OUT_UNIT, OUT_UNIT)])

        plsc.subcore_barrier()


@functools.partial(jax.jit, static_argnames=())
def _sc_aggregate(xs, src, dst):
    mesh = plsc.VectorSubcoreMesh(core_axis_name="c", subcore_axis_name="s")
    f = pl.kernel(
        _sc_body,
        out_type=jax.ShapeDtypeStruct((NC, N_REL, N_NODES, DH), jnp.float32),
        mesh=mesh,
        scratch_types=[
            pltpu.VMEM_SHARED((ACC_ROWS, DH), jnp.float32),
            pltpu.VMEM((E_TILE,), jnp.int32),
            pltpu.VMEM((CHUNKS_PER_TILE, CHUNK), jnp.int32),
            pltpu.VMEM((2, CHUNK, DH), jnp.float32),
            pltpu.VMEM((8, DH), jnp.float32),
            pltpu.SemaphoreType.DMA((2,)),
        ],
        compiler_params=pltpu.CompilerParams(use_tc_tiling_on_sc=False),
    )
    return f(xs, src, dst)


def _wcat_body(coeff_ref, w2d_ref, wself_ref, o_ref):
    wall = jnp.dot(coeff_ref[...], w2d_ref[...],
                   preferred_element_type=jnp.float32,
                   precision=jax.lax.Precision.HIGHEST)
    o_ref[: N_REL * D_IN, :] = wall.reshape(N_REL * D_IN, D_OUT)
    o_ref[N_REL * D_IN:, :] = wself_ref[...]


def _combine_wcat(coeff_mat, w2d, W_self):
    return pl.pallas_call(
        _wcat_body,
        out_shape=jax.ShapeDtypeStruct(((N_REL + 1) * D_IN, D_OUT), jnp.float32),
    )(coeff_mat, w2d, W_self)


def _z0(i):
    return i * 0


BM = 400


def _tc_body(s_ref, x_ref, wcat_ref, bias_ref, o_ref):
    parts = []
    for r in range(N_REL):
        cnt = s_ref[0, r, :, CH:CH + 1]
        sm = jnp.concatenate(
            [s_ref[0, r, :, :CH], s_ref[1, r, :, :CH]], axis=1)
        parts.append(jnp.where(cnt > 0, sm / jnp.maximum(cnt, 1.0), 0.0))
    parts.append(x_ref[...])
    xin = jnp.concatenate(parts, axis=1)
    acc = jnp.dot(xin, wcat_ref[...], preferred_element_type=jnp.float32)
    o_ref[...] = jnp.maximum(acc + bias_ref[...], 0.0)


def _tc_combine(s, x, wcat, bias2d):
    grid = (N_NODES // BM,)
    return pl.pallas_call(
        _tc_body,
        grid=grid,
        in_specs=[
            pl.BlockSpec((NC, N_REL, BM, DH),
                         lambda i: (_z0(i), _z0(i), i, _z0(i))),
            pl.BlockSpec((BM, D_IN), lambda i: (i, _z0(i))),
            pl.BlockSpec(((N_REL + 1) * D_IN, D_OUT),
                         lambda i: (_z0(i), _z0(i))),
            pl.BlockSpec((1, D_OUT), lambda i: (_z0(i), _z0(i))),
        ],
        out_specs=pl.BlockSpec((BM, D_OUT), lambda i: (i, _z0(i))),
        out_shape=jax.ShapeDtypeStruct((N_NODES, D_OUT), jnp.float32),
    )(s, x, wcat, bias2d)


def kernel(x, edge_index_r0, edge_index_r1, edge_index_r2, edge_index_r3,
           w, coeff_mat, W_self, bias):
    x = x.astype(jnp.float32)
    ones = jnp.ones((N_NODES, 1), jnp.float32)
    zpad = jnp.zeros((N_NODES, DH - CH - 1), jnp.float32)
    xs = jnp.stack([
        jnp.concatenate([x[:, :CH], ones, zpad], axis=1),
        jnp.concatenate([x[:, CH:], ones, zpad], axis=1),
    ])

    srcs, dsts = [], []
    for e in (edge_index_r0, edge_index_r1, edge_index_r2, edge_index_r3):
        src = e[0].astype(jnp.int32)
        dst = e[1].astype(jnp.int32)
        srcs.append(jnp.concatenate(
            [src, jnp.zeros((E_PAD - E,), jnp.int32)]))
        dsts.append(jnp.concatenate(
            [dst, jnp.full((E_PAD - E,), N_NODES, jnp.int32)]))
    src = jnp.stack(srcs)
    dst = jnp.stack(dsts)

    s = _sc_aggregate(xs, src, dst)

    w2d = w.astype(jnp.float32).reshape(w.shape[0], D_IN * D_OUT)
    wcat = _combine_wcat(coeff_mat.astype(jnp.float32), w2d,
                         W_self.astype(jnp.float32))
    bias2d = bias.astype(jnp.float32).reshape(1, D_OUT)
    return _tc_combine(s, x, wcat, bias2d)
